# baseline (speedup 1.0000x reference)
import functools

import jax
import jax.numpy as jnp
from jax import lax
from jax.experimental import pallas as pl
from jax.experimental.pallas import tpu as pltpu
from jax.experimental.pallas import tpu_sc as plsc

_B = 128
_V = 100000
_MARGIN = 1.0

_NC = 2
_NS = 16
_NW = _NC * _NS
_RPW = _B // _NW
_CH = 20000
_CPR = _V // _CH
_NCH = _RPW * _CPR
_U = 10
_VECS = _CH // (16 * _U)

_NEG = float("-inf")


def _body(x_hbm, gold_hbm, out_hbm, buf0, buf1, gold_v, out_v, sem0, sem1):
    w = lax.axis_index("s") * _NC + lax.axis_index("c")
    wbase = pl.multiple_of(w * (_RPW * _V), 8)
    bufs = (buf0, buf1)
    sems = (sem0, sem1)
    iota = lax.iota(jnp.int32, 16)

    def start(t):
        off = pl.multiple_of(wbase + t * _CH, 8)
        return pltpu.async_copy(x_hbm.at[pl.ds(off, _CH)], bufs[t % 2],
                                sems[t % 2])

    cps = [None] * _NCH
    cps[0] = start(0)
    cps[1] = start(1)
    pltpu.sync_copy(gold_hbm, gold_v)

    def chunk_body(buf, i, carry):
        new = []
        base = i * (16 * _U)
        for u in range(_U):
            v = buf[pl.ds(base + u * 16, 16)]
            m1, m2 = carry[2 * u], carry[2 * u + 1]
            new.append(jnp.maximum(m1, v))
            new.append(jnp.maximum(m2, jnp.minimum(m1, v)))
        return tuple(new)

    partial = jnp.float32(0.0)
    for r in range(_RPW):
        row = w * _RPW + r
        gold_col = plsc.load_gather(gold_v, [jnp.broadcast_to(row, (16,))])

        acc = [jnp.full((16,), _NEG, dtype=jnp.float32)] * (2 * _U)
        g = jnp.zeros((16,), dtype=jnp.float32)
        for c in range(_CPR):
            t = r * _CPR + c
            cps[t].wait()
            buf = bufs[t % 2]
            accs = lax.fori_loop(0, _VECS, functools.partial(chunk_body, buf),
                                 tuple(acc))
            acc = list(accs)
            cb = c * _CH
            inb = jnp.logical_and(gold_col >= cb, gold_col < cb + _CH)
            loc = jnp.clip(gold_col - cb, 0, _CH - 1)
            gv = plsc.load_gather(buf, [loc])
            g = jnp.where(inb, gv, g)
            if t + 2 < _NCH:
                cps[t + 2] = start(t + 2)

        m1, m2 = acc[0], acc[1]
        for u in range(1, _U):
            a1, a2 = acc[2 * u], acc[2 * u + 1]
            m2 = jnp.maximum(jnp.maximum(m2, a2), jnp.minimum(m1, a1))
            m1 = jnp.maximum(m1, a1)
        top1 = jnp.max(m1)
        ffs = plsc.all_reduce_ffs(m1 == top1)
        masked = jnp.where(iota == ffs, _NEG, m1)
        top2 = jnp.maximum(jnp.max(masked), jnp.max(m2))

        g = jnp.max(g)
        loss = jnp.where(g == top1,
                         jnp.maximum(jnp.float32(_MARGIN) + top2 - top1, 0.0),
                         jnp.float32(_MARGIN) + top1 - g)
        partial = partial + loss

    out_v[...] = jnp.where(iota == 0, partial, jnp.float32(0.0))
    pltpu.sync_copy(out_v, out_hbm.at[w])


_sc_call = pl.kernel(
    _body,
    out_type=jax.ShapeDtypeStruct((_NW, 16), jnp.float32),
    mesh=plsc.VectorSubcoreMesh(core_axis_name="c", subcore_axis_name="s"),
    compiler_params=pltpu.CompilerParams(needs_layout_passes=False),
    scratch_types=[
        pltpu.VMEM((_CH,), jnp.float32),
        pltpu.VMEM((_CH,), jnp.float32),
        pltpu.VMEM((_B,), jnp.int32),
        pltpu.VMEM((16,), jnp.float32),
        pltpu.SemaphoreType.DMA,
        pltpu.SemaphoreType.DMA,
    ],
)


@jax.jit
def kernel(x, gold):
    partials = _sc_call(x.reshape(-1), gold)
    return jnp.sum(partials) / jnp.float32(_B)


# speedup vs baseline: 1.6801x; 1.6801x over previous
import jax
import jax.numpy as jnp
from jax import lax
from jax.experimental import pallas as pl
from jax.experimental.pallas import tpu as pltpu
from jax.experimental.pallas import tpu_sc as plsc

_B = 128
_V = 100000
_MARGIN = 1.0

_NT = 782
_TPW = 391
_KT = 23
_NCHK = 17
_CC = _KT * 128
_HALF = _TPW * 128
_G = 4
_NVEC = _KT * 8
_VALID_LAST = 178

_NEG = float("-inf")
_POS = float("inf")


def _body(x_hbm, gold_hbm, out_hbm, buf0, buf1, gold_v, stage_v, nb_v,
          out_v, shared, sem0, sem1):
    c = lax.axis_index("c")
    s = lax.axis_index("s")
    grp = c * 8 + s // 2
    h = s % 2
    g8 = grp * 8
    iota = lax.iota(jnp.int32, 16)
    bufs = (buf0, buf1)
    sems = (sem0, sem1)

    r0 = pl.multiple_of(g8, 8)
    cbase = pl.multiple_of(h * _HALF, 128)

    def start(k):
        coff = pl.multiple_of(cbase + k * _CC, 128)
        return pltpu.async_copy(x_hbm.at[pl.ds(r0, 8), pl.ds(coff, _CC)],
                                bufs[k % 2], sems[k % 2])

    cps = [None] * _NCHK
    cps[0] = start(0)
    cps[1] = start(1)
    pltpu.sync_copy(gold_hbm, gold_v)

    base16 = pl.multiple_of((g8 // 16) * 16, 16)
    g16f = gold_v[pl.ds(base16, 16)].astype(jnp.float32)
    lane0 = g8 - base16
    gold_i = [jnp.max(jnp.where(iota == lane0 + r, g16f, -1.0))
              .astype(jnp.int32) for r in range(8)]

    bad = h == 1

    def mk_body(buf, last):
        def body(i, carry):
            accs = list(carry)
            for j in range(_G):
                idx = i * _G + j
                col = idx * 16
                if last:
                    pen = jnp.where(
                        jnp.logical_and(bad, idx >= _VALID_LAST),
                        jnp.float32(_NEG), jnp.float32(_POS))
                for r in range(8):
                    v = buf[r, pl.ds(col, 16)]
                    if last:
                        v = jnp.minimum(v, pen)
                    a1, a2 = accs[2 * r], accs[2 * r + 1]
                    accs[2 * r + 1] = jnp.maximum(a2, jnp.minimum(a1, v))
                    accs[2 * r] = jnp.maximum(a1, v)
            return tuple(accs)
        return body

    acc = [jnp.full((16,), _NEG, dtype=jnp.float32)] * 16
    gsc = [jnp.float32(_NEG)] * 8
    for k in range(_NCHK):
        cps[k].wait()
        buf = bufs[k % 2]
        acc = list(lax.fori_loop(0, _NVEC // _G,
                                 mk_body(buf, k == _NCHK - 1), tuple(acc)))
        cstart = cbase + k * _CC
        for r in range(8):
            loc = gold_i[r] - cstart
            inb = jnp.logical_and(loc >= 0, loc < _CC)
            locc = jnp.clip(loc, 0, _CC - 1)
            al = (locc // 16) * 16
            va = buf[r, pl.ds(al, 16)]
            gv = jnp.max(jnp.where(iota == locc - al, va, jnp.float32(_NEG)))
            gsc[r] = jnp.where(inb, gv, gsc[r])
        if k + 2 < _NCHK:
            cps[k + 2] = start(k + 2)

    top1v = jnp.full((16,), _NEG, dtype=jnp.float32)
    top2v = jnp.full((16,), _NEG, dtype=jnp.float32)
    gv = jnp.full((16,), _NEG, dtype=jnp.float32)
    for r in range(8):
        m1, m2 = acc[2 * r], acc[2 * r + 1]
        t1 = jnp.max(m1)
        ffs = plsc.all_reduce_ffs(m1 == t1)
        masked = jnp.where(iota == ffs, jnp.float32(_NEG), m1)
        t2 = jnp.maximum(jnp.max(masked), jnp.max(m2))
        top1v = jnp.where(iota == r, t1, top1v)
        top2v = jnp.where(iota == r, t2, top2v)
        gv = jnp.where(iota == r, gsc[r], gv)

    stage_v[pl.ds(0, 16)] = top1v
    stage_v[pl.ds(16, 16)] = top2v
    stage_v[pl.ds(32, 16)] = gv
    pltpu.sync_copy(stage_v, shared.at[pl.ds(s * 128, 48)])
    plsc.subcore_barrier()

    @pl.when(h == 0)
    def _merge():
        pltpu.sync_copy(shared.at[pl.ds((s + 1) * 128, 48)], nb_v)
        b1 = nb_v[pl.ds(0, 16)]
        b2 = nb_v[pl.ds(16, 16)]
        bg = nb_v[pl.ds(32, 16)]
        m1 = jnp.maximum(top1v, b1)
        m2 = jnp.maximum(jnp.maximum(top2v, b2), jnp.minimum(top1v, b1))
        gm = jnp.maximum(gv, bg)
        loss = jnp.where(gm == m1,
                         jnp.maximum(jnp.float32(_MARGIN) + m2 - m1, 0.0),
                         jnp.float32(_MARGIN) + m1 - gm)
        out_v[...] = jnp.where(iota < 8, loss, jnp.float32(0.0))
        pltpu.sync_copy(out_v, out_hbm.at[pl.ds(grp * 128, 16)])


_sc_call = pl.kernel(
    _body,
    name="ranking_loss_sc",
    out_type=jax.ShapeDtypeStruct((16 * 128,), jnp.float32),
    mesh=plsc.VectorSubcoreMesh(core_axis_name="c", subcore_axis_name="s"),
    compiler_params=pltpu.CompilerParams(needs_layout_passes=False,
                                         use_tc_tiling_on_sc=True),
    scratch_types=[
        pltpu.VMEM((8, _CC), jnp.float32),
        pltpu.VMEM((8, _CC), jnp.float32),
        pltpu.VMEM((_B,), jnp.int32),
        pltpu.VMEM((48,), jnp.float32),
        pltpu.VMEM((48,), jnp.float32),
        pltpu.VMEM((16,), jnp.float32),
        pltpu.VMEM_SHARED((16 * 128,), jnp.float32),
        pltpu.SemaphoreType.DMA,
        pltpu.SemaphoreType.DMA,
    ],
)


@jax.jit
def kernel(x, gold):
    partials = _sc_call(x, gold)
    losses = partials.reshape(16, 128)[:, :16]
    return jnp.sum(losses) / jnp.float32(_B)


# speedup vs baseline: 3.0657x; 1.8246x over previous
import jax
import jax.numpy as jnp
from jax import lax
from jax.experimental import pallas as pl
from jax.experimental.pallas import tpu as pltpu
from jax.experimental.pallas import tpu_sc as plsc

_B = 128
_V = 100000
_MARGIN = 1.0

_TPW = 391
_KT = 23
_NCHK = 17
_CV = _KT * 8
_SPW = _TPW * 8
_VLAST = _V - _CV
_G = 4

_NEG = float("-inf")
_POS = float("inf")


def _body(xt_hbm, gold_hbm, out_hbm, buf0, buf1, gold_v, stage_v, allbuf,
          grow_v, out_v, shared, sem0, sem1, gsem):
    c = lax.axis_index("c")
    s = lax.axis_index("s")
    w = c * 16 + s
    iota = lax.iota(jnp.int32, 16)
    bufs = (buf0, buf1)
    sems = (sem0, sem1)

    wbase = w * _SPW

    def start(k):
        voff_u = wbase + k * _CV
        voff = jnp.minimum(voff_u, _VLAST) if k == _NCHK - 1 else voff_u
        voff = pl.multiple_of(voff, 8)
        return pltpu.async_copy(xt_hbm.at[pl.ds(voff, _CV), :],
                                bufs[k % 2], sems[k % 2])

    cps = [None] * _NCHK
    cps[0] = start(0)
    cps[1] = start(1)
    pltpu.sync_copy(gold_hbm, gold_v)

    def mk_body(buf, last):
        def body(i, carry):
            accs = list(carry)
            for j in range(_G):
                vloc = i * _G + j
                if last:
                    voff_u = wbase + (_NCHK - 1) * _CV
                    voff = jnp.minimum(voff_u, _VLAST)
                    pen = jnp.where(voff + vloc >= voff_u,
                                    jnp.float32(_POS), jnp.float32(_NEG))
                for seg in range(8):
                    v = buf[vloc, pl.ds(seg * 16, 16)]
                    if last:
                        v = jnp.minimum(v, pen)
                    a1, a2 = accs[2 * seg], accs[2 * seg + 1]
                    accs[2 * seg + 1] = jnp.maximum(a2, jnp.minimum(a1, v))
                    accs[2 * seg] = jnp.maximum(a1, v)
            return tuple(accs)
        return body

    acc = [jnp.full((16,), _NEG, dtype=jnp.float32)] * 16
    for k in range(_NCHK):
        cps[k].wait()
        acc = list(lax.fori_loop(0, _CV // _G,
                                 mk_body(bufs[k % 2], k == _NCHK - 1),
                                 tuple(acc)))
        if k + 2 < _NCHK:
            cps[k + 2] = start(k + 2)

    for seg in range(8):
        stage_v[pl.ds(seg * 32, 16)] = acc[2 * seg]
        stage_v[pl.ds(seg * 32 + 16, 16)] = acc[2 * seg + 1]
    pltpu.sync_copy(stage_v, shared.at[pl.ds(s * 256, 256)])
    plsc.subcore_barrier()

    @pl.when(s < 8)
    def _merge():
        pltpu.sync_copy(shared, allbuf)
        soff = s * 32
        m1 = jnp.full((16,), _NEG, dtype=jnp.float32)
        m2 = jnp.full((16,), _NEG, dtype=jnp.float32)
        for j in range(16):
            a1 = allbuf[pl.ds(j * 256 + soff, 16)]
            a2 = allbuf[pl.ds(j * 256 + soff + 16, 16)]
            m2 = jnp.maximum(jnp.maximum(m2, a2), jnp.minimum(m1, a1))
            m1 = jnp.maximum(m1, a1)
        idxv = gold_v[pl.ds(s * 16, 16)]
        pltpu.async_copy(xt_hbm.at[idxv], grow_v, gsem).wait()
        g = jnp.full((16,), _NEG, dtype=jnp.float32)
        for l in range(16):
            rowv = grow_v[l, pl.ds(s * 16, 16)]
            g = jnp.where(iota == l, rowv, g)
        out_v[pl.ds(0, 16)] = m1
        out_v[pl.ds(16, 16)] = m2
        out_v[pl.ds(32, 16)] = g
        obase = pl.multiple_of((c * 8 + s) * 128, 128)
        pltpu.sync_copy(out_v, out_hbm.at[pl.ds(obase, 48)])


_sc_call = pl.kernel(
    _body,
    name="ranking_loss_sc",
    out_type=jax.ShapeDtypeStruct((16 * 128,), jnp.float32),
    mesh=plsc.VectorSubcoreMesh(core_axis_name="c", subcore_axis_name="s"),
    compiler_params=pltpu.CompilerParams(needs_layout_passes=False,
                                         use_tc_tiling_on_sc=True),
    scratch_types=[
        pltpu.VMEM((_CV, _B), jnp.float32),
        pltpu.VMEM((_CV, _B), jnp.float32),
        pltpu.VMEM((_B,), jnp.int32),
        pltpu.VMEM((256,), jnp.float32),
        pltpu.VMEM((4096,), jnp.float32),
        pltpu.VMEM((16, _B), jnp.float32),
        pltpu.VMEM((48,), jnp.float32),
        pltpu.VMEM_SHARED((4096,), jnp.float32),
        pltpu.SemaphoreType.DMA,
        pltpu.SemaphoreType.DMA,
        pltpu.SemaphoreType.DMA,
    ],
)


@jax.jit
def kernel(x, gold):
    partials = _sc_call(x.T, gold)
    p = partials.reshape(16, 128)[:, :48].reshape(2, 8, 3, 16)
    a, b = p[0], p[1]
    m1 = jnp.maximum(a[:, 0], b[:, 0])
    m2 = jnp.maximum(jnp.maximum(a[:, 1], b[:, 1]),
                     jnp.minimum(a[:, 0], b[:, 0]))
    g = jnp.maximum(a[:, 2], b[:, 2])
    loss = jnp.where(g == m1,
                     jnp.maximum(jnp.float32(_MARGIN) + m2 - m1, 0.0),
                     jnp.float32(_MARGIN) + m1 - g)
    return jnp.sum(loss) / jnp.float32(_B)


# speedup vs baseline: 3.2305x; 1.0538x over previous
import jax
import jax.numpy as jnp
from jax import lax
from jax.experimental import pallas as pl
from jax.experimental.pallas import tpu as pltpu
from jax.experimental.pallas import tpu_sc as plsc

_B = 128
_V = 100000
_MARGIN = 1.0

_TPW = 391
_KT = 23
_NCHK = 17
_CV = _KT * 8
_SPW = _TPW * 8
_VLAST = _V - _CV
_G = 8
_NBUF = 3

_NEG = float("-inf")
_POS = float("inf")


def _body(xt_hbm, gold_hbm, out_hbm, buf0, buf1, buf2, gold_v, stage_v,
          allbuf, grow_v, out_v, shared, sem0, sem1, sem2, gsem):
    c = lax.axis_index("c")
    s = lax.axis_index("s")
    w = c * 16 + s
    iota = lax.iota(jnp.int32, 16)
    bufs = (buf0, buf1, buf2)
    sems = (sem0, sem1, sem2)

    wbase = w * _SPW

    def start(k):
        voff_u = wbase + k * _CV
        voff = jnp.minimum(voff_u, _VLAST) if k == _NCHK - 1 else voff_u
        voff = pl.multiple_of(voff, 8)
        return pltpu.async_copy(xt_hbm.at[pl.ds(voff, _CV), :],
                                bufs[k % _NBUF], sems[k % _NBUF])

    cps = [None] * _NCHK
    for k in range(_NBUF):
        cps[k] = start(k)
    pltpu.sync_copy(gold_hbm, gold_v)

    voff_u_last = wbase + (_NCHK - 1) * _CV
    thresh = voff_u_last - jnp.minimum(voff_u_last, _VLAST)

    def mk_body(buf, last):
        def body(i, carry):
            accs = list(carry)
            for j in range(_G):
                vloc = i * _G + j
                if last:
                    pen = jnp.where(vloc >= thresh,
                                    jnp.float32(_POS), jnp.float32(_NEG))
                for seg in range(8):
                    v = buf[vloc, pl.ds(seg * 16, 16)]
                    if last:
                        v = jnp.minimum(v, pen)
                    a1, a2 = accs[2 * seg], accs[2 * seg + 1]
                    accs[2 * seg + 1] = jnp.maximum(a2, jnp.minimum(a1, v))
                    accs[2 * seg] = jnp.maximum(a1, v)
            return tuple(accs)
        return body

    acc = [jnp.full((16,), _NEG, dtype=jnp.float32)] * 16
    for k in range(_NCHK):
        cps[k].wait()
        acc = list(lax.fori_loop(0, _CV // _G,
                                 mk_body(bufs[k % _NBUF], k == _NCHK - 1),
                                 tuple(acc)))
        if k + _NBUF < _NCHK:
            cps[k + _NBUF] = start(k + _NBUF)

    for seg in range(8):
        stage_v[pl.ds(seg * 32, 16)] = acc[2 * seg]
        stage_v[pl.ds(seg * 32 + 16, 16)] = acc[2 * seg + 1]
    pltpu.sync_copy(stage_v, shared.at[pl.ds(s * 256, 256)])
    plsc.subcore_barrier()

    @pl.when(s < 8)
    def _merge():
        pltpu.sync_copy(shared, allbuf)
        soff = s * 32
        m1 = jnp.full((16,), _NEG, dtype=jnp.float32)
        m2 = jnp.full((16,), _NEG, dtype=jnp.float32)
        for j in range(16):
            a1 = allbuf[pl.ds(j * 256 + soff, 16)]
            a2 = allbuf[pl.ds(j * 256 + soff + 16, 16)]
            m2 = jnp.maximum(jnp.maximum(m2, a2), jnp.minimum(m1, a1))
            m1 = jnp.maximum(m1, a1)
        idxv = gold_v[pl.ds(s * 16, 16)]
        pltpu.async_copy(xt_hbm.at[idxv], grow_v, gsem).wait()
        g = jnp.full((16,), _NEG, dtype=jnp.float32)
        for l in range(16):
            rowv = grow_v[l, pl.ds(s * 16, 16)]
            g = jnp.where(iota == l, rowv, g)
        out_v[pl.ds(0, 16)] = m1
        out_v[pl.ds(16, 16)] = m2
        out_v[pl.ds(32, 16)] = g
        obase = pl.multiple_of((c * 8 + s) * 128, 128)
        pltpu.sync_copy(out_v, out_hbm.at[pl.ds(obase, 48)])


_sc_call = pl.kernel(
    _body,
    name="ranking_loss_sc",
    out_type=jax.ShapeDtypeStruct((16 * 128,), jnp.float32),
    mesh=plsc.VectorSubcoreMesh(core_axis_name="c", subcore_axis_name="s"),
    compiler_params=pltpu.CompilerParams(needs_layout_passes=False,
                                         use_tc_tiling_on_sc=True),
    scratch_types=[
        pltpu.VMEM((_CV, _B), jnp.float32),
        pltpu.VMEM((_CV, _B), jnp.float32),
        pltpu.VMEM((_CV, _B), jnp.float32),
        pltpu.VMEM((_B,), jnp.int32),
        pltpu.VMEM((256,), jnp.float32),
        pltpu.VMEM((4096,), jnp.float32),
        pltpu.VMEM((16, _B), jnp.float32),
        pltpu.VMEM((48,), jnp.float32),
        pltpu.VMEM_SHARED((4096,), jnp.float32),
        pltpu.SemaphoreType.DMA,
        pltpu.SemaphoreType.DMA,
        pltpu.SemaphoreType.DMA,
        pltpu.SemaphoreType.DMA,
    ],
)


@jax.jit
def kernel(x, gold):
    partials = _sc_call(x.T, gold)
    p = partials.reshape(16, 128)[:, :48].reshape(2, 8, 3, 16)
    a, b = p[0], p[1]
    m1 = jnp.maximum(a[:, 0], b[:, 0])
    m2 = jnp.maximum(jnp.maximum(a[:, 1], b[:, 1]),
                     jnp.minimum(a[:, 0], b[:, 0]))
    g = jnp.maximum(a[:, 2], b[:, 2])
    loss = jnp.where(g == m1,
                     jnp.maximum(jnp.float32(_MARGIN) + m2 - m1, 0.0),
                     jnp.float32(_MARGIN) + m1 - g)
    return jnp.sum(loss) / jnp.float32(_B)
